# baseline (speedup 1.0000x reference)
import jax
import jax.numpy as jnp
from jax import lax
from jax.experimental import pallas as pl
from jax.experimental.pallas import tpu as pltpu
from jax.experimental.pallas import tpu_sc as plsc

N = 10000
F = 128
FH = F // 2
C = 40

NC = 2
NS = 16
NW = NC * NS
CH = 128
NBUF = 2
NCH = -(-162 // NBUF) * NBUF
TOT_E = NS * NCH * CH
NCH_DEG = TOT_E // (NW * CH)
ACC_ROWS = 10240
STRIPE = ACC_ROWS // NS
DUMMY = 10200

_mesh = plsc.VectorSubcoreMesh(core_axis_name="c", subcore_axis_name="s")


def _fill(buf, width, value):
    v = jnp.full((16,), value, jnp.float32)

    def row(i, _):
        for j in range(width // 16):
            buf[i, pl.ds(j * 16, 16)] = v
        return 0

    lax.fori_loop(0, 128, row, 0)


OUTER = NCH // NBUF


def _agg_body(y_hbm, src_hbm, dst_hbm, out_hbm, src_l, dst_l, rows, zbuf, acc,
              gsems, ssems):
    c = lax.axis_index("c")
    s = lax.axis_index("s")

    pltpu.sync_copy(src_hbm.at[s], src_l)
    pltpu.sync_copy(dst_hbm.at[s], dst_l)

    _fill(zbuf, FH, 0.0)
    for b in range(STRIPE // 128):
        pltpu.sync_copy(zbuf, acc.at[pl.ds(s * STRIPE + b * 128, 128)])
    plsc.subcore_barrier()

    yh = y_hbm.at[c]

    def prime(b, _):
        pltpu.async_copy(yh.at[src_l.at[b]], rows.at[b], gsems.at[b])
        return 0

    lax.fori_loop(0, NBUF, prime, 0)

    def outer(o, _):
        base = o * NBUF

        def fire_scatter(b, _):
            j = base + b
            pltpu.make_async_copy(yh.at[src_l.at[j]], rows.at[b],
                                  gsems.at[b]).wait()
            pltpu.async_copy(rows.at[b], acc.at[dst_l.at[j]], ssems.at[b],
                             add=True)
            return 0

        def refill(b, _):
            j = base + b
            pltpu.make_async_copy(rows.at[b], acc.at[dst_l.at[j]],
                                  ssems.at[b]).wait()
            nxt = j + NBUF

            @pl.when(nxt < NCH)
            def _():
                pltpu.async_copy(yh.at[src_l.at[nxt]], rows.at[b], gsems.at[b])

            return 0

        lax.fori_loop(0, NBUF, fire_scatter, 0)
        lax.fori_loop(0, NBUF, refill, 0)
        return 0

    lax.fori_loop(0, OUTER, outer, 0)
    plsc.subcore_barrier()

    pltpu.sync_copy(acc.at[pl.ds(s * STRIPE, STRIPE)],
                    out_hbm.at[c, pl.ds(s * STRIPE, STRIPE)])


_agg = pl.kernel(
    _agg_body,
    out_type=jax.ShapeDtypeStruct((NC, ACC_ROWS, FH), jnp.float32),
    mesh=_mesh,
    scratch_types=[
        pltpu.VMEM((NCH, CH), jnp.int32),
        pltpu.VMEM((NCH, CH), jnp.int32),
        pltpu.VMEM((NBUF, CH, FH), jnp.float32),
        pltpu.VMEM((128, FH), jnp.float32),
        pltpu.VMEM_SHARED((ACC_ROWS, FH), jnp.float32),
        pltpu.SemaphoreType.DMA((NBUF,)),
        pltpu.SemaphoreType.DMA((NBUF,)),
    ],
    compiler_params=pltpu.CompilerParams(use_tc_tiling_on_sc=False),
)


def _deg_body(dst_hbm, out_hbm, dst_l, acc1):
    c = lax.axis_index("c")
    s = lax.axis_index("s")
    wid = s * NC + c

    pltpu.sync_copy(dst_hbm.at[wid], dst_l)

    zv = jnp.zeros((16,), jnp.float32)

    def zrow(i, _):
        acc1[pl.ds(i * 16, 16)] = zv
        return 0

    lax.fori_loop(0, ACC_ROWS // 16, zrow, 0)

    ones = jnp.full((16,), 1.0, jnp.float32)

    def chunk(j, _):
        for k in range(CH // 16):
            idx = dst_l[j, pl.ds(k * 16, 16)]
            plsc.addupdate_scatter(acc1, [idx], ones)
        return 0

    lax.fori_loop(0, NCH_DEG, chunk, 0)

    pltpu.sync_copy(acc1, out_hbm.at[wid])


_deg = pl.kernel(
    _deg_body,
    out_type=jax.ShapeDtypeStruct((NW, ACC_ROWS), jnp.float32),
    mesh=_mesh,
    scratch_types=[
        pltpu.VMEM((NCH_DEG, CH), jnp.int32),
        pltpu.VMEM((ACC_ROWS,), jnp.float32),
    ],
    compiler_params=pltpu.CompilerParams(use_tc_tiling_on_sc=False,
                                         needs_layout_passes=False),
)


_BM = 512
_NBLK = ACC_ROWS // _BM


def _prep_body(deg_ref, x_ref, dinv_ref, y_ref):
    deg = jnp.sum(deg_ref[...], axis=0)[:, None]
    dinv = lax.rsqrt(deg)
    dinv_ref[...] = dinv
    y = x_ref[...] * dinv
    y_ref[0] = y[:, :FH]
    y_ref[1] = y[:, FH:]


def _tc_prep(deg_parts, x):
    return pl.pallas_call(
        _prep_body,
        grid=(_NBLK,),
        in_specs=[
            pl.BlockSpec((NW, _BM), lambda i: (0, i)),
            pl.BlockSpec((_BM, F), lambda i: (i, 0)),
        ],
        out_specs=[
            pl.BlockSpec((_BM, 1), lambda i: (i, 0)),
            pl.BlockSpec((NC, _BM, FH), lambda i: (0, i, 0)),
        ],
        out_shape=[
            jax.ShapeDtypeStruct((N, 1), jnp.float32),
            jax.ShapeDtypeStruct((NC, N, FH), jnp.float32),
        ],
    )(deg_parts, x)


def _layer_body(s_ref, dinv_ref, w_ref, b_ref, y_ref):
    dinv = dinv_ref[...]
    t = jnp.concatenate([s_ref[0], s_ref[1]], axis=1) * dinv
    h = jnp.dot(t, w_ref[...], preferred_element_type=jnp.float32)
    y = jnp.maximum(h + b_ref[...], 0.0) * dinv
    y_ref[0] = y[:, :FH]
    y_ref[1] = y[:, FH:]


def _tc_layer(s_parts, dinv, w, b):
    return pl.pallas_call(
        _layer_body,
        grid=(_NBLK,),
        in_specs=[
            pl.BlockSpec((NC, _BM, FH), lambda i: (0, i, 0)),
            pl.BlockSpec((_BM, 1), lambda i: (i, 0)),
            pl.BlockSpec((F, F), lambda i: (0, 0)),
            pl.BlockSpec((1, F), lambda i: (0, 0)),
        ],
        out_specs=pl.BlockSpec((NC, _BM, FH), lambda i: (0, i, 0)),
        out_shape=jax.ShapeDtypeStruct((NC, N, FH), jnp.float32),
    )(s_parts, dinv, w, b)


def _final_body(s_ref, dinv_ref, w_ref, b_ref, o_ref):
    t = jnp.concatenate([s_ref[0], s_ref[1]], axis=1) * dinv_ref[...]
    z = jnp.dot(t, w_ref[...], preferred_element_type=jnp.float32) + b_ref[...]
    col = lax.broadcasted_iota(jnp.int32, (_BM, F), 1)
    zm = jnp.where(col < C, z, -jnp.inf)
    m = jnp.max(zm, axis=1, keepdims=True)
    lse = m + jnp.log(jnp.sum(jnp.exp(zm - m), axis=1, keepdims=True))
    o_ref[...] = z - lse


def _tc_final(s_parts, dinv, w, b):
    return pl.pallas_call(
        _final_body,
        grid=(_NBLK,),
        in_specs=[
            pl.BlockSpec((NC, _BM, FH), lambda i: (0, i, 0)),
            pl.BlockSpec((_BM, 1), lambda i: (i, 0)),
            pl.BlockSpec((F, F), lambda i: (0, 0)),
            pl.BlockSpec((1, F), lambda i: (0, 0)),
        ],
        out_specs=pl.BlockSpec((_BM, F), lambda i: (i, 0)),
        out_shape=jax.ShapeDtypeStruct((N, F), jnp.float32),
    )(s_parts, dinv, w, b)


def kernel(x, edge_index, W1, b1, W2, b2, W3, b3, W4, b4, W5, b5):
    src = edge_index[0]
    dst = edge_index[1]
    loop = jnp.arange(N, dtype=src.dtype)
    pad = TOT_E - (src.shape[0] + N)
    src2 = jnp.concatenate([src, loop, jnp.zeros((pad,), src.dtype)])
    dst2 = jnp.concatenate([dst, loop, jnp.full((pad,), DUMMY, dst.dtype)])
    src_r = src2.reshape(NS, NCH, CH)
    dst_r = dst2.reshape(NS, NCH, CH)
    dst_r32 = dst2.reshape(NW, NCH_DEG, CH)

    deg_parts = _deg(dst_r32)
    dinv, y = _tc_prep(deg_parts, x)

    for w, b in ((W1, b1), (W2, b2), (W3, b3), (W4, b4)):
        s_parts = _agg(y, src_r, dst_r)
        y = _tc_layer(s_parts, dinv, w, b.reshape(1, F))

    s_parts = _agg(y, src_r, dst_r)
    w5p = jnp.pad(W5, ((0, 0), (0, F - C)))
    b5p = jnp.pad(b5, (0, F - C)).reshape(1, F)
    out = _tc_final(s_parts, dinv, w5p, b5p)
    return out[:, :C]


# speedup vs baseline: 1.2283x; 1.2283x over previous
import jax
import jax.numpy as jnp
from jax import lax
from jax.experimental import pallas as pl
from jax.experimental.pallas import tpu as pltpu
from jax.experimental.pallas import tpu_sc as plsc

N = 10000
F = 128
FH = F // 2
C = 40

NC = 2
NS = 16
NW = NC * NS
CH = 128
NBUF = 3
NCH = -(-162 // NBUF) * NBUF
TOT_E = NS * NCH * CH
NCH_DEG = TOT_E // (NW * CH)
ACC_ROWS = 10240
STRIPE = ACC_ROWS // NS
DUMMY = 10200

_mesh = plsc.VectorSubcoreMesh(core_axis_name="c", subcore_axis_name="s")


def _fill(buf, width, value):
    v = jnp.full((16,), value, jnp.float32)

    def row(i, _):
        for j in range(width // 16):
            buf[i, pl.ds(j * 16, 16)] = v
        return 0

    lax.fori_loop(0, 128, row, 0)


OUTER = NCH // NBUF


def _agg_body(y_hbm, src_hbm, dst_hbm, out_hbm, src_l, dst_l, rows, zbuf, acc,
              gsems, ssems):
    c = lax.axis_index("c")
    s = lax.axis_index("s")

    pltpu.sync_copy(src_hbm.at[s], src_l)
    pltpu.sync_copy(dst_hbm.at[s], dst_l)

    _fill(zbuf, FH, 0.0)
    for b in range(STRIPE // 128):
        pltpu.sync_copy(zbuf, acc.at[pl.ds(s * STRIPE + b * 128, 128)])
    plsc.subcore_barrier()

    yh = y_hbm.at[c]

    def prime(b, _):
        pltpu.async_copy(yh.at[src_l.at[b]], rows.at[b], gsems.at[b])
        return 0

    lax.fori_loop(0, NBUF, prime, 0)

    def outer(o, _):
        base = o * NBUF

        def fire_scatter(b, _):
            j = base + b
            pltpu.make_async_copy(yh.at[src_l.at[j]], rows.at[b],
                                  gsems.at[b]).wait()
            pltpu.async_copy(rows.at[b], acc.at[dst_l.at[j]], ssems.at[b],
                             add=True)
            return 0

        def refill(b, _):
            j = base + b
            pltpu.make_async_copy(rows.at[b], acc.at[dst_l.at[j]],
                                  ssems.at[b]).wait()
            nxt = j + NBUF

            @pl.when(nxt < NCH)
            def _():
                pltpu.async_copy(yh.at[src_l.at[nxt]], rows.at[b], gsems.at[b])

            return 0

        lax.fori_loop(0, NBUF, fire_scatter, 0)
        lax.fori_loop(0, NBUF, refill, 0)
        return 0

    lax.fori_loop(0, OUTER, outer, 0)
    plsc.subcore_barrier()

    pltpu.sync_copy(acc.at[pl.ds(s * STRIPE, STRIPE)],
                    out_hbm.at[c, pl.ds(s * STRIPE, STRIPE)])


_agg = pl.kernel(
    _agg_body,
    out_type=jax.ShapeDtypeStruct((NC, ACC_ROWS, FH), jnp.float32),
    mesh=_mesh,
    scratch_types=[
        pltpu.VMEM((NCH, CH), jnp.int32),
        pltpu.VMEM((NCH, CH), jnp.int32),
        pltpu.VMEM((NBUF, CH, FH), jnp.float32),
        pltpu.VMEM((128, FH), jnp.float32),
        pltpu.VMEM_SHARED((ACC_ROWS, FH), jnp.float32),
        pltpu.SemaphoreType.DMA((NBUF,)),
        pltpu.SemaphoreType.DMA((NBUF,)),
    ],
    compiler_params=pltpu.CompilerParams(use_tc_tiling_on_sc=False),
)


def _deg_body(dst_hbm, out_hbm, dst_l, acc1):
    c = lax.axis_index("c")
    s = lax.axis_index("s")
    wid = s * NC + c

    pltpu.sync_copy(dst_hbm.at[wid], dst_l)

    zv = jnp.zeros((16,), jnp.float32)

    def zrow(i, _):
        acc1[pl.ds(i * 16, 16)] = zv
        return 0

    lax.fori_loop(0, ACC_ROWS // 16, zrow, 0)

    ones = jnp.full((16,), 1.0, jnp.float32)

    def chunk(j, _):
        for k in range(CH // 16):
            idx = dst_l[j, pl.ds(k * 16, 16)]
            plsc.addupdate_scatter(acc1, [idx], ones)
        return 0

    lax.fori_loop(0, NCH_DEG, chunk, 0)

    pltpu.sync_copy(acc1, out_hbm.at[wid])


_deg = pl.kernel(
    _deg_body,
    out_type=jax.ShapeDtypeStruct((NW, ACC_ROWS), jnp.float32),
    mesh=_mesh,
    scratch_types=[
        pltpu.VMEM((NCH_DEG, CH), jnp.int32),
        pltpu.VMEM((ACC_ROWS,), jnp.float32),
    ],
    compiler_params=pltpu.CompilerParams(use_tc_tiling_on_sc=False,
                                         needs_layout_passes=False),
)


def _prep_body(deg_ref, x_ref, dinv_ref, y_ref):
    deg = jnp.sum(deg_ref[...], axis=0)[:N, None]
    dinv = lax.rsqrt(deg)
    dinv_ref[...] = dinv
    y = x_ref[...] * dinv
    y_ref[0] = y[:, :FH]
    y_ref[1] = y[:, FH:]


def _tc_prep(deg_parts, x):
    return pl.pallas_call(
        _prep_body,
        out_shape=[
            jax.ShapeDtypeStruct((N, 1), jnp.float32),
            jax.ShapeDtypeStruct((NC, N, FH), jnp.float32),
        ],
    )(deg_parts, x)


def _layer_body(s_ref, dinv_ref, w_ref, b_ref, y_ref):
    dinv = dinv_ref[...]
    t = jnp.concatenate([s_ref[0, :N], s_ref[1, :N]], axis=1) * dinv
    h = jnp.dot(t, w_ref[...], preferred_element_type=jnp.float32)
    y = jnp.maximum(h + b_ref[...], 0.0) * dinv
    y_ref[0] = y[:, :FH]
    y_ref[1] = y[:, FH:]


def _tc_layer(s_parts, dinv, w, b):
    return pl.pallas_call(
        _layer_body,
        out_shape=jax.ShapeDtypeStruct((NC, N, FH), jnp.float32),
    )(s_parts, dinv, w, b)


def _final_body(s_ref, dinv_ref, w_ref, b_ref, o_ref):
    t = jnp.concatenate([s_ref[0, :N], s_ref[1, :N]], axis=1) * dinv_ref[...]
    z = jnp.dot(t, w_ref[...], preferred_element_type=jnp.float32) + b_ref[...]
    col = lax.broadcasted_iota(jnp.int32, (N, F), 1)
    zm = jnp.where(col < C, z, -jnp.inf)
    m = jnp.max(zm, axis=1, keepdims=True)
    lse = m + jnp.log(jnp.sum(jnp.exp(zm - m), axis=1, keepdims=True))
    o_ref[...] = z - lse


def _tc_final(s_parts, dinv, w, b):
    return pl.pallas_call(
        _final_body,
        out_shape=jax.ShapeDtypeStruct((N, F), jnp.float32),
    )(s_parts, dinv, w, b)


def kernel(x, edge_index, W1, b1, W2, b2, W3, b3, W4, b4, W5, b5):
    src = edge_index[0]
    dst = edge_index[1]
    loop = jnp.arange(N, dtype=src.dtype)
    pad = TOT_E - (src.shape[0] + N)
    src2 = jnp.concatenate([src, loop, jnp.zeros((pad,), src.dtype)])
    dst2 = jnp.concatenate([dst, loop, jnp.full((pad,), DUMMY, dst.dtype)])
    src_r = src2.reshape(NS, NCH, CH)
    dst_r = dst2.reshape(NS, NCH, CH)
    dst_r32 = dst2.reshape(NW, NCH_DEG, CH)

    deg_parts = _deg(dst_r32)
    dinv, y = _tc_prep(deg_parts, x)

    for w, b in ((W1, b1), (W2, b2), (W3, b3), (W4, b4)):
        s_parts = _agg(y, src_r, dst_r)
        y = _tc_layer(s_parts, dinv, w, b.reshape(1, F))

    s_parts = _agg(y, src_r, dst_r)
    w5p = jnp.pad(W5, ((0, 0), (0, F - C)))
    b5p = jnp.pad(b5, (0, F - C)).reshape(1, F)
    out = _tc_final(s_parts, dinv, w5p, b5p)
    return out[:, :C]


# speedup vs baseline: 1.2315x; 1.0026x over previous
import jax
import jax.numpy as jnp
from jax import lax
from jax.experimental import pallas as pl
from jax.experimental.pallas import tpu as pltpu
from jax.experimental.pallas import tpu_sc as plsc

N = 10000
F = 128
FH = F // 2
C = 40

NC = 2
NS = 16
NW = NC * NS
CH = 128
NBUF = 3
NCH = -(-162 // NBUF) * NBUF
TOT_E = NS * NCH * CH
NCH_DEG = TOT_E // (NW * CH)
ACC_ROWS = 10240
STRIPE = ACC_ROWS // NS
DUMMY = 10200

_mesh = plsc.VectorSubcoreMesh(core_axis_name="c", subcore_axis_name="s")


def _fill(buf, width, value):
    v = jnp.full((16,), value, jnp.float32)

    def row(i, _):
        for j in range(width // 16):
            buf[i, pl.ds(j * 16, 16)] = v
        return 0

    lax.fori_loop(0, 128, row, 0)


OUTER = NCH // NBUF


def _agg_body(y_hbm, src_hbm, dst_hbm, out_hbm, src_l, dst_l, rows, zbuf, acc,
              gsems, ssems):
    c = lax.axis_index("c")
    s = lax.axis_index("s")

    pltpu.sync_copy(src_hbm.at[s], src_l)
    pltpu.sync_copy(dst_hbm.at[s], dst_l)

    _fill(zbuf, FH, 0.0)
    for b in range(STRIPE // 128):
        pltpu.sync_copy(zbuf, acc.at[pl.ds(s * STRIPE + b * 128, 128)])
    plsc.subcore_barrier()

    yh = y_hbm.at[c]

    def prime(b, _):
        pltpu.async_copy(yh.at[src_l.at[b]], rows.at[b], gsems.at[b])
        return 0

    lax.fori_loop(0, NBUF, prime, 0)

    def outer(o, _):
        base = o * NBUF

        def fire_scatter(b, _):
            j = base + b
            pltpu.make_async_copy(yh.at[src_l.at[j]], rows.at[b],
                                  gsems.at[b]).wait()
            pltpu.async_copy(rows.at[b], acc.at[dst_l.at[j]], ssems.at[b],
                             add=True)
            return 0

        def refill(b, _):
            j = base + b
            pltpu.make_async_copy(rows.at[b], acc.at[dst_l.at[j]],
                                  ssems.at[b]).wait()
            nxt = j + NBUF

            @pl.when(nxt < NCH)
            def _():
                pltpu.async_copy(yh.at[src_l.at[nxt]], rows.at[b], gsems.at[b])

            return 0

        lax.fori_loop(0, NBUF, fire_scatter, 0)
        lax.fori_loop(0, NBUF, refill, 0)
        return 0

    lax.fori_loop(0, OUTER, outer, 0)
    plsc.subcore_barrier()

    pltpu.sync_copy(acc.at[pl.ds(s * STRIPE, STRIPE)],
                    out_hbm.at[c, pl.ds(s * STRIPE, STRIPE)])


_agg = pl.kernel(
    _agg_body,
    out_type=jax.ShapeDtypeStruct((NC, ACC_ROWS, FH), jnp.float32),
    mesh=_mesh,
    scratch_types=[
        pltpu.VMEM((NCH, CH), jnp.int32),
        pltpu.VMEM((NCH, CH), jnp.int32),
        pltpu.VMEM((NBUF, CH, FH), jnp.float32),
        pltpu.VMEM((128, FH), jnp.float32),
        pltpu.VMEM_SHARED((ACC_ROWS, FH), jnp.float32),
        pltpu.SemaphoreType.DMA((NBUF,)),
        pltpu.SemaphoreType.DMA((NBUF,)),
    ],
    compiler_params=pltpu.CompilerParams(use_tc_tiling_on_sc=False),
)


def _deg_body(dst_hbm, out_hbm, dst_l, acc1):
    c = lax.axis_index("c")
    s = lax.axis_index("s")
    wid = s * NC + c

    pltpu.sync_copy(dst_hbm.at[wid], dst_l)

    zv = jnp.zeros((16,), jnp.float32)

    def zrow(i, _):
        acc1[pl.ds(i * 16, 16)] = zv
        return 0

    lax.fori_loop(0, ACC_ROWS // 16, zrow, 0)

    ones = jnp.full((16,), 1.0, jnp.float32)

    def chunk(j, _):
        for k in range(CH // 16):
            idx = dst_l[j, pl.ds(k * 16, 16)]
            plsc.addupdate_scatter(acc1, [idx], ones)
        return 0

    lax.fori_loop(0, NCH_DEG, chunk, 0)

    pltpu.sync_copy(acc1, out_hbm.at[wid])


_deg = pl.kernel(
    _deg_body,
    out_type=jax.ShapeDtypeStruct((NW, ACC_ROWS), jnp.float32),
    mesh=_mesh,
    scratch_types=[
        pltpu.VMEM((NCH_DEG, CH), jnp.int32),
        pltpu.VMEM((ACC_ROWS,), jnp.float32),
    ],
    compiler_params=pltpu.CompilerParams(use_tc_tiling_on_sc=False,
                                         needs_layout_passes=False),
)


def _prep_body(deg_ref, x_ref, dinv_ref, y_ref):
    deg = jnp.sum(deg_ref[...], axis=0)[:N, None]
    dinv = lax.rsqrt(deg)
    dinv_ref[...] = dinv
    y = x_ref[...] * dinv
    y_ref[0] = y[:, :FH]
    y_ref[1] = y[:, FH:]


def _tc_prep(deg_parts, x):
    return pl.pallas_call(
        _prep_body,
        out_shape=[
            jax.ShapeDtypeStruct((N, 1), jnp.float32),
            jax.ShapeDtypeStruct((NC, N, FH), jnp.float32),
        ],
    )(deg_parts, x)


def _layer_body(s_ref, dinv_ref, w_ref, b_ref, y_ref):
    dinv = dinv_ref[...]
    t = jnp.concatenate([s_ref[0, :N], s_ref[1, :N]], axis=1) * dinv
    h = jnp.dot(t, w_ref[...], preferred_element_type=jnp.float32)
    y = jnp.maximum(h + b_ref[...], 0.0) * dinv
    y_ref[0] = y[:, :FH]
    y_ref[1] = y[:, FH:]


def _tc_layer(s_parts, dinv, w, b):
    return pl.pallas_call(
        _layer_body,
        out_shape=jax.ShapeDtypeStruct((NC, N, FH), jnp.float32),
    )(s_parts, dinv, w, b)


def _final_body(s_ref, dinv_ref, w_ref, b_ref, o_ref):
    t = jnp.concatenate([s_ref[0, :N], s_ref[1, :N]], axis=1) * dinv_ref[...]
    z = jnp.dot(t, w_ref[...], preferred_element_type=jnp.float32) + b_ref[...]
    col = lax.broadcasted_iota(jnp.int32, (N, F), 1)
    zm = jnp.where(col < C, z, -jnp.inf)
    m = jnp.max(zm, axis=1, keepdims=True)
    lse = m + jnp.log(jnp.sum(jnp.exp(zm - m), axis=1, keepdims=True))
    o_ref[...] = (z - lse)[:, :C]


def _tc_final(s_parts, dinv, w, b):
    return pl.pallas_call(
        _final_body,
        out_shape=jax.ShapeDtypeStruct((N, C), jnp.float32),
    )(s_parts, dinv, w, b)


def kernel(x, edge_index, W1, b1, W2, b2, W3, b3, W4, b4, W5, b5):
    src = edge_index[0]
    dst = edge_index[1]
    loop = jnp.arange(N, dtype=src.dtype)
    pad = TOT_E - (src.shape[0] + N)
    src2 = jnp.concatenate([src, loop, jnp.zeros((pad,), src.dtype)])
    dst2 = jnp.concatenate([dst, loop, jnp.full((pad,), DUMMY, dst.dtype)])
    src_r = src2.reshape(NS, NCH, CH)
    dst_r = dst2.reshape(NS, NCH, CH)
    dst_r32 = dst2.reshape(NW, NCH_DEG, CH)

    deg_parts = _deg(dst_r32)
    dinv, y = _tc_prep(deg_parts, x)

    for w, b in ((W1, b1), (W2, b2), (W3, b3), (W4, b4)):
        s_parts = _agg(y, src_r, dst_r)
        y = _tc_layer(s_parts, dinv, w, b.reshape(1, F))

    s_parts = _agg(y, src_r, dst_r)
    w5p = jnp.pad(W5, ((0, 0), (0, F - C)))
    b5p = jnp.pad(b5, (0, F - C)).reshape(1, F)
    return _tc_final(s_parts, dinv, w5p, b5p)
